# Initial kernel scaffold; baseline (speedup 1.0000x reference)
import jax
import jax.numpy as jnp
from jax.experimental import pallas as pl


def kernel(x, params, edge_weight, edge_index, idx_train, labels):
    raise NotImplementedError("write your pallas kernel here")


# speedup vs baseline: 3.7492x; 3.7492x over previous
import functools

import jax
import jax.numpy as jnp
from jax import lax
from jax.experimental import pallas as pl
from jax.experimental.pallas import tpu as pltpu
from jax.experimental.pallas import tpu_sc as plsc

_N = 10000
_E = 320000
_NFEAT = 128
_NLAYERS = 4
_NHID1 = 64
_NCLASS = 40
_NHID2 = 64
_NHEADS = 8
_ALPHA = 0.2
_NTRAIN = 1000

_NC = 2
_NS = 16
_NW = _NC * _NS
_EPW = _E // _NW
_CH = 80
_NCHUNK = _EPW // _CH
_ZR = 208
_RPT = 624

_BLK = 1000
_GRID = _N // _BLK


def _sc_spmm(H, src, dst, wvals=None, s1=None, s2=None, nh=1):
    gat = wvals is None
    W = H.shape[1]
    WA = W + 16 if gat else W
    Wh = W // nh

    mesh = plsc.VectorSubcoreMesh(core_axis_name="c", subcore_axis_name="s",
                                  num_cores=_NC, num_subcores=_NS)

    scratch = [
        pltpu.VMEM((_CH,), jnp.int32),
        pltpu.VMEM((_CH,), jnp.int32),
        pltpu.VMEM((_CH, W), jnp.float32),
        pltpu.VMEM((nh, _CH + 16), jnp.float32),
        pltpu.VMEM_SHARED((_N, WA), jnp.float32),
        pltpu.SemaphoreType.DMA,
    ]
    if gat:
        scratch.append(pltpu.VMEM((_CH, WA), jnp.float32))
        for _ in range(2 * nh):
            scratch.append(pltpu.VMEM((_CH,), jnp.float32))

    def body(*refs):
        if gat:
            (h_hbm, src_hbm, dst_hbm, *rest) = refs
            s1_hbms = rest[:nh]
            s2_hbms = rest[nh:2 * nh]
            (out_hbm, srcb, dstb, rows_g, wbuf, acc, sem,
             rows_s, *sgbufs) = rest[2 * nh:]
            s1g = sgbufs[:nh]
            s2g = sgbufs[nh:]
        else:
            (h_hbm, src_hbm, dst_hbm, w_hbm, out_hbm,
             srcb, dstb, rows_g, wbuf, acc, sem) = refs
            rows_s = rows_g

        cid = lax.axis_index("c")
        sid = lax.axis_index("s")
        wid = cid * _NS + sid

        def zfill(i, _):
            for f in range(WA // 16):
                rows_s[i, pl.ds(f * 16, 16)] = jnp.zeros((16,), jnp.float32)
            return 0
        lax.fori_loop(0, _CH, zfill, 0)

        rbase = pl.multiple_of(sid * _RPT, 8)
        for t in range(7):
            pltpu.sync_copy(rows_s, acc.at[pl.ds(rbase + t * _CH, _CH)])
        pltpu.sync_copy(rows_s.at[pl.ds(0, 64)],
                        acc.at[pl.ds(rbase + 7 * _CH, 64)])

        @pl.when(sid == _NS - 1)
        def _tail_zero():
            pltpu.sync_copy(rows_s.at[pl.ds(0, 16)],
                            acc.at[pl.ds(_NS * _RPT, 16)])

        plsc.subcore_barrier()

        def chunk(i, _):
            base = pl.multiple_of(wid * _EPW + i * _CH, 8)
            pltpu.sync_copy(src_hbm.at[pl.ds(base, _CH)], srcb)
            pltpu.sync_copy(dst_hbm.at[pl.ds(base, _CH)], dstb)
            pltpu.async_copy(h_hbm.at[dstb], rows_g, sem).wait()

            if gat:
                for hh in range(nh):
                    pltpu.async_copy(s1_hbms[hh].at[srcb], s1g[hh], sem).wait()
                    pltpu.async_copy(s2_hbms[hh].at[dstb], s2g[hh], sem).wait()
                for v in range(_CH // 16):
                    for hh in range(nh):
                        av = s1g[hh][pl.ds(v * 16, 16)]
                        bv = s2g[hh][pl.ds(v * 16, 16)]
                        cv = av + bv
                        ev = jnp.exp(-jnp.where(cv > 0, cv, _ALPHA * cv))
                        wbuf[hh, pl.ds(v * 16, 16)] = ev
            else:
                pltpu.sync_copy(w_hbm.at[pl.ds(base, _CH)],
                                wbuf.at[0, pl.ds(0, _CH)])

            lane = lax.iota(jnp.int32, 16)

            def scale(j, _):
                ws = [wbuf[hh, pl.ds(j, 16)][0] for hh in range(nh)]
                for hh in range(nh):
                    for f in range(Wh // 16):
                        c0 = hh * Wh + f * 16
                        rows_s[j, pl.ds(c0, 16)] = (
                            rows_g[j, pl.ds(c0, 16)] * ws[hh])
                if gat:
                    ev = jnp.where(lane == 0, ws[0], 0.0)
                    if nh == 2:
                        ev = jnp.where(lane == 1, ws[1], ev)
                    rows_s[j, pl.ds(W, 16)] = ev
                return 0
            lax.fori_loop(0, _CH, scale, 0)

            pltpu.sync_copy(rows_s, acc.at[srcb], add=True)
            return 0
        lax.fori_loop(0, _NCHUNK, chunk, 0)

        plsc.subcore_barrier()

        for t in range(3):
            pltpu.sync_copy(acc.at[pl.ds(rbase + t * _ZR, _ZR)],
                            out_hbm.at[cid, pl.ds(rbase + t * _ZR, _ZR)])

        @pl.when(sid == _NS - 1)
        def _tail_copy():
            pltpu.sync_copy(acc.at[pl.ds(_NS * _RPT, 16)],
                            out_hbm.at[cid, pl.ds(_NS * _RPT, 16)])

    call = pl.kernel(
        body,
        out_type=jax.ShapeDtypeStruct((_NC, _N, WA), jnp.float32),
        mesh=mesh,
        scratch_types=scratch,
        compiler_params=pltpu.CompilerParams(use_tc_tiling_on_sc=False,
                                             needs_layout_passes=False),
    )
    if gat:
        return call(H, src, dst, *s1, *s2)
    return call(H, src, dst, wvals)


def _tc_snowball_mm(x, s_list, b_list, Wk, wout):
    nin = len(s_list)

    def body(*refs):
        x_ref = refs[0]
        s_refs = refs[1:1 + nin]
        b_refs = refs[1 + nin:1 + 2 * nin]
        w_ref = refs[1 + 2 * nin]
        o_ref = refs[-1]
        acc = jnp.dot(x_ref[...], w_ref[:_NFEAT],
                      preferred_element_type=jnp.float32)
        for j in range(nin):
            bj = jnp.tanh(s_refs[j][0] + s_refs[j][1] + b_refs[j][...])
            acc = acc + jnp.dot(
                bj, w_ref[_NFEAT + _NHID1 * j:_NFEAT + _NHID1 * (j + 1)],
                preferred_element_type=jnp.float32)
        o_ref[...] = acc

    in_specs = [pl.BlockSpec((_BLK, _NFEAT), lambda i: (i, 0))]
    for _ in range(nin):
        in_specs.append(pl.BlockSpec((2, _BLK, _NHID1), lambda i: (0, i, 0)))
    for _ in range(nin):
        in_specs.append(pl.BlockSpec((1, _NHID1), lambda i: (0, 0)))
    in_specs.append(pl.BlockSpec(Wk.shape, lambda i: (0, 0)))

    return pl.pallas_call(
        body,
        grid=(_GRID,),
        in_specs=in_specs,
        out_specs=pl.BlockSpec((_BLK, wout), lambda i: (i, 0)),
        out_shape=jax.ShapeDtypeStruct((_N, wout), jnp.float32),
    )(x, *s_list, *b_list, Wk)


def _tc_gat_proj(x, w_all, a_l, a_r):
    def body(x_ref, w_ref, al_ref, ar_ref, hp_ref, s1_ref, s2_ref):
        xb = x_ref[...]
        c1, c2 = [], []
        for p in range(4):
            hb = jnp.dot(xb, w_ref[:, 128 * p:128 * (p + 1)],
                         preferred_element_type=jnp.float32)
            hp_ref[p] = hb
            for hh in range(2):
                h = 2 * p + hh
                hv = hb[:, _NHID2 * hh:_NHID2 * (hh + 1)]
                c1.append(jnp.sum(hv * al_ref[h][None, :], axis=1))
                c2.append(jnp.sum(hv * ar_ref[h][None, :], axis=1))
        s1_ref[...] = jnp.stack(c1, axis=1)
        s2_ref[...] = jnp.stack(c2, axis=1)

    return pl.pallas_call(
        body,
        grid=(_GRID,),
        in_specs=[
            pl.BlockSpec((_BLK, _NFEAT), lambda i: (i, 0)),
            pl.BlockSpec((_NFEAT, _NHID2 * _NHEADS), lambda i: (0, 0)),
            pl.BlockSpec((_NHEADS, _NHID2), lambda i: (0, 0)),
            pl.BlockSpec((_NHEADS, _NHID2), lambda i: (0, 0)),
        ],
        out_specs=[
            pl.BlockSpec((4, _BLK, 128), lambda i: (0, i, 0)),
            pl.BlockSpec((_BLK, _NHEADS), lambda i: (i, 0)),
            pl.BlockSpec((_BLK, _NHEADS), lambda i: (i, 0)),
        ],
        out_shape=[
            jax.ShapeDtypeStruct((4, _N, 128), jnp.float32),
            jax.ShapeDtypeStruct((_N, _NHEADS), jnp.float32),
            jax.ShapeDtypeStruct((_N, _NHEADS), jnp.float32),
        ],
    )(x, w_all, a_l, a_r)


def _tc_gat_out(accs, oa_w, oa_l, oa_r):
    def body(*refs):
        a_refs = refs[:4]
        oaw_ref, oal_ref, oar_ref, g_ref, so_ref = refs[4:]
        g = jnp.zeros((_BLK, _NCLASS + 8), jnp.float32)
        for p in range(4):
            ab = a_refs[p]
            for hh in range(2):
                h = 2 * p + hh
                hp = (ab[0, :, _NHID2 * hh:_NHID2 * (hh + 1)]
                      + ab[1, :, _NHID2 * hh:_NHID2 * (hh + 1)])
                rs = ab[0, :, 128 + hh] + ab[1, :, 128 + hh]
                z0 = hp / (rs + 1e-15)[:, None]
                z0 = jnp.where(z0 > 0, z0, jnp.exp(z0) - 1.0)
                g = g + jnp.dot(
                    z0, oaw_ref[_NHID2 * h:_NHID2 * (h + 1)],
                    preferred_element_type=jnp.float32)
        g_ref[...] = g
        so_ref[...] = jnp.stack(
            [jnp.sum(g * oal_ref[...], axis=1),
             jnp.sum(g * oar_ref[...], axis=1)], axis=1)

    return pl.pallas_call(
        body,
        grid=(_GRID,),
        in_specs=[pl.BlockSpec((2, _BLK, 144), lambda i: (0, i, 0))] * 4
        + [
            pl.BlockSpec((_NHID2 * _NHEADS, _NCLASS + 8), lambda i: (0, 0)),
            pl.BlockSpec((1, _NCLASS + 8), lambda i: (0, 0)),
            pl.BlockSpec((1, _NCLASS + 8), lambda i: (0, 0)),
        ],
        out_specs=[
            pl.BlockSpec((_BLK, _NCLASS + 8), lambda i: (i, 0)),
            pl.BlockSpec((_BLK, 2), lambda i: (i, 0)),
        ],
        out_shape=[
            jax.ShapeDtypeStruct((_N, _NCLASS + 8), jnp.float32),
            jax.ShapeDtypeStruct((_N, 2), jnp.float32),
        ],
    )(*accs, oa_w, oa_l, oa_r)


def _tc_final(s4, out_b, go, idx_t, lab_f):
    NCP = _NCLASS + 8

    def body(s4_ref, ob_ref, go_ref, it_ref, lf_ref,
             y_ref, z_ref, l1_ref, l2_ref, cl_ref,
             ytr, ztr, ltr, clacc):
        i = pl.program_id(0)

        @pl.when(i == 0)
        def _init():
            ytr[...] = jnp.zeros_like(ytr)
            ztr[...] = jnp.zeros_like(ztr)
            ltr[...] = jnp.zeros_like(ltr)
            clacc[...] = jnp.zeros_like(clacc)

        col = lax.broadcasted_iota(jnp.int32, (_BLK, NCP), 1)
        mask = col < _NCLASS

        y_blk = s4_ref[0] + s4_ref[1] + ob_ref[...]
        rs = go_ref[0, :, NCP] + go_ref[1, :, NCP]
        zpre = (go_ref[0, :, :NCP] + go_ref[1, :, :NCP]) / (rs + 1e-15)[:, None]
        z_blk = jnp.where(zpre > 0, zpre, jnp.exp(zpre) - 1.0)
        y_ref[...] = y_blk
        z_ref[...] = z_blk

        ym = jnp.where(mask, y_blk, -1e30)
        my = jnp.max(ym, axis=1, keepdims=True)
        lsey = my + jnp.log(jnp.sum(jnp.exp(ym - my), axis=1, keepdims=True))
        logpy = ym - lsey
        zm = jnp.where(mask, z_blk, -1e30)
        mz = jnp.max(zm, axis=1, keepdims=True)
        pz = jnp.exp(zm - mz)
        pz = pz / jnp.sum(pz, axis=1, keepdims=True)
        clacc[...] += jnp.sum(jnp.where(mask, pz * (-logpy), 0.0)
                              ).reshape(1, 1)

        rows = i * _BLK + lax.broadcasted_iota(jnp.int32, (_NTRAIN, _BLK), 1)
        mf = jnp.where(it_ref[...] == rows, 1.0, 0.0)
        ytr[...] += jnp.dot(mf, y_blk, preferred_element_type=jnp.float32)
        ztr[...] += jnp.dot(mf, z_blk, preferred_element_type=jnp.float32)
        ltr[...] += jnp.dot(mf, lf_ref[...], preferred_element_type=jnp.float32)

        @pl.when(i == _GRID - 1)
        def _fin():
            tcol = lax.broadcasted_iota(jnp.int32, (_NTRAIN, NCP), 1)
            tmask = tcol < _NCLASS
            oh = jnp.where((tcol.astype(jnp.float32) == ltr[...]) & tmask,
                           1.0, 0.0)

            def ce(t_ref):
                tm = jnp.where(tmask, t_ref[...], -1e30)
                mt = jnp.max(tm, axis=1, keepdims=True)
                lse = mt + jnp.log(
                    jnp.sum(jnp.exp(tm - mt), axis=1, keepdims=True))
                lp = tm - lse
                return -jnp.sum(oh * lp) / _NTRAIN

            l1_ref[...] = ce(ytr).reshape(1, 1)
            l2_ref[...] = ce(ztr).reshape(1, 1)
            cl_ref[...] = clacc[...] / _N

    return pl.pallas_call(
        body,
        grid=(_GRID,),
        in_specs=[
            pl.BlockSpec((2, _BLK, NCP), lambda i: (0, i, 0)),
            pl.BlockSpec((1, NCP), lambda i: (0, 0)),
            pl.BlockSpec((2, _BLK, NCP + 16), lambda i: (0, i, 0)),
            pl.BlockSpec((_NTRAIN, 1), lambda i: (0, 0)),
            pl.BlockSpec((_BLK, 1), lambda i: (i, 0)),
        ],
        out_specs=[
            pl.BlockSpec((_BLK, NCP), lambda i: (i, 0)),
            pl.BlockSpec((_BLK, NCP), lambda i: (i, 0)),
            pl.BlockSpec((1, 1), lambda i: (0, 0)),
            pl.BlockSpec((1, 1), lambda i: (0, 0)),
            pl.BlockSpec((1, 1), lambda i: (0, 0)),
        ],
        out_shape=[
            jax.ShapeDtypeStruct((_N, NCP), jnp.float32),
            jax.ShapeDtypeStruct((_N, NCP), jnp.float32),
            jax.ShapeDtypeStruct((1, 1), jnp.float32),
            jax.ShapeDtypeStruct((1, 1), jnp.float32),
            jax.ShapeDtypeStruct((1, 1), jnp.float32),
        ],
        scratch_shapes=[
            pltpu.VMEM((_NTRAIN, NCP), jnp.float32),
            pltpu.VMEM((_NTRAIN, NCP), jnp.float32),
            pltpu.VMEM((_NTRAIN, 1), jnp.float32),
            pltpu.VMEM((1, 1), jnp.float32),
        ],
    )(s4, out_b, go, idx_t, lab_f)


def kernel(x, params, edge_weight, edge_index, idx_train, labels):
    src = edge_index[0]
    dst = edge_index[1]

    s_parts = []
    b_list = []
    for k in range(_NLAYERS):
        p_k = _tc_snowball_mm(x, s_parts, b_list,
                              params['snow_W_%d' % k], _NHID1)
        s_parts.append(_sc_spmm(p_k, src, dst, wvals=edge_weight))
        b_list.append(params['snow_b_%d' % k].reshape(1, _NHID1))

    w_out = jnp.pad(params['out_W'], ((0, 0), (0, 8)))
    p_out = _tc_snowball_mm(x, s_parts, b_list, w_out, _NCLASS + 8)
    s4 = _sc_spmm(p_out, src, dst, wvals=edge_weight)

    w_all = jnp.concatenate(
        [params['att_W_%d' % h] for h in range(_NHEADS)], axis=1)
    a_l = jnp.stack([params['att_a_%d' % h][0, :_NHID2]
                     for h in range(_NHEADS)])
    a_r = jnp.stack([params['att_a_%d' % h][0, _NHID2:]
                     for h in range(_NHEADS)])
    hp, s1, s2 = _tc_gat_proj(x, w_all, a_l, a_r)
    s1t = s1.T
    s2t = s2.T
    accs = [
        _sc_spmm(hp[p], src, dst,
                 s1=[s1t[2 * p], s1t[2 * p + 1]],
                 s2=[s2t[2 * p], s2t[2 * p + 1]], nh=2)
        for p in range(4)
    ]

    oa_w = jnp.pad(params['outatt_W'], ((0, 0), (0, 8)))
    oa_l = jnp.pad(params['outatt_a'][0, :_NCLASS], (0, 8))[None]
    oa_r = jnp.pad(params['outatt_a'][0, _NCLASS:], (0, 8))[None]
    g, so = _tc_gat_out(accs, oa_w, oa_l, oa_r)
    sot = so.T
    go = _sc_spmm(g, src, dst, s1=[sot[0]], s2=[sot[1]], nh=1)

    out_b = jnp.pad(params['out_b'], (0, 8))[None]
    y48, z48, l1, l2, cl = _tc_final(
        s4, out_b, go,
        idx_train.reshape(_NTRAIN, 1),
        labels.astype(jnp.float32).reshape(_N, 1))

    y = y48[:, :_NCLASS]
    z = z48[:, :_NCLASS]
    l1s = l1[0, 0]
    l2s = l2[0, 0]
    cls = cl[0, 0]
    return (y, z, l1s, l2s,
            l1s + jnp.float32(0.05) * cls, l2s + jnp.float32(0.05) * cls)


# speedup vs baseline: 6.2930x; 1.6785x over previous
import functools

import jax
import jax.numpy as jnp
from jax import lax
from jax.experimental import pallas as pl
from jax.experimental.pallas import tpu as pltpu
from jax.experimental.pallas import tpu_sc as plsc

_N = 10000
_E = 320000
_NFEAT = 128
_NLAYERS = 4
_NHID1 = 64
_NCLASS = 40
_NHID2 = 64
_NHEADS = 8
_ALPHA = 0.2
_NTRAIN = 1000

_NC = 2
_NS = 16
_NW = _NC * _NS
_EPW = _E // _NW
_CH = 80
_NCHUNK = _EPW // _CH
_ZR = 208
_RPT = 624

_BLK = 1000
_GRID = _N // _BLK


def _sc_spmm(H, src, dst, wvals=None, s1=None, s2=None, nh=1):
    gat = wvals is None
    W = H.shape[1]
    WA = W + 16 if gat else W
    Wh = W // nh
    CH = 40 if W > 64 else 80
    NCHUNK = _EPW // CH
    NBLK = (NCHUNK + 3) // 4
    ZB = 104

    mesh = plsc.VectorSubcoreMesh(core_axis_name="c", subcore_axis_name="s",
                                  num_cores=_NC, num_subcores=_NS)

    nw = 4 if not gat else 1
    scratch = (
        [pltpu.VMEM((CH,), jnp.int32) for _ in range(8)]
        + [pltpu.VMEM((CH, W), jnp.float32) for _ in range(2)]
        + [pltpu.VMEM((CH, WA), jnp.float32) for _ in range(2)]
        + [pltpu.VMEM((nh, CH + 16), jnp.float32) for _ in range(nw)]
        + [pltpu.VMEM((ZB, WA), jnp.float32)]
        + [pltpu.VMEM_SHARED((_N, WA), jnp.float32)]
        + [pltpu.SemaphoreType.DMA for _ in range(5)]
        + ([pltpu.VMEM((CH + 16,), jnp.float32) for _ in range(4 * nh)]
           if gat else [])
    )

    def body(*refs):
        if gat:
            (h_hbm, src_hbm, dst_hbm, *rest) = refs
            s1_hbms = rest[:nh]
            s2_hbms = rest[nh:2 * nh]
            rest = rest[2 * nh:]
            out_hbm = rest[0]
            rest = rest[1:]
        else:
            (h_hbm, src_hbm, dst_hbm, w_hbm, out_hbm, *rest) = refs
        SRC = rest[0:4]
        DST = rest[4:8]
        RG = rest[8:10]
        RS = rest[10:12]
        WB = rest[12:12 + nw]
        zbuf = rest[12 + nw]
        acc = rest[13 + nw]
        gsem = rest[14 + nw]
        ISEM = rest[15 + nw:17 + nw]
        SSEM = rest[17 + nw:19 + nw]
        if gat:
            sg = rest[19 + nw:]
            S1G = [sg[2 * hh:2 * hh + 2] for hh in range(nh)]
            S2G = [sg[2 * nh + 2 * hh:2 * nh + 2 * hh + 2] for hh in range(nh)]

        cid = lax.axis_index("c")
        sid = lax.axis_index("s")
        wid = cid * _NS + sid

        def zfill(i, _):
            for f in range(WA // 16):
                zbuf[i, pl.ds(f * 16, 16)] = jnp.zeros((16,), jnp.float32)
            return 0
        lax.fori_loop(0, ZB, zfill, 0)

        rbase = pl.multiple_of(sid * _RPT, 8)
        for t in range(6):
            pltpu.sync_copy(zbuf, acc.at[pl.ds(rbase + t * ZB, ZB)])

        @pl.when(sid == _NS - 1)
        def _tail_zero():
            pltpu.sync_copy(zbuf.at[pl.ds(0, 16)],
                            acc.at[pl.ds(_NS * _RPT, 16)])

        plsc.subcore_barrier()

        def base_of(i):
            return pl.multiple_of(wid * _EPW + i * CH, 8)

        def issue_idx(i, q, par):
            b = base_of(i)
            pltpu.async_copy(src_hbm.at[pl.ds(b, CH)], SRC[q], ISEM[par])
            pltpu.async_copy(dst_hbm.at[pl.ds(b, CH)], DST[q], ISEM[par])
            if not gat:
                pltpu.async_copy(w_hbm.at[pl.ds(b, CH)],
                                 WB[q].at[0, pl.ds(0, CH)], ISEM[par])

        def wait_idx(q, par):
            pltpu.make_async_copy(src_hbm.at[pl.ds(0, CH)], SRC[q],
                                  ISEM[par]).wait()
            pltpu.make_async_copy(dst_hbm.at[pl.ds(0, CH)], DST[q],
                                  ISEM[par]).wait()
            if not gat:
                pltpu.make_async_copy(w_hbm.at[pl.ds(0, CH)],
                                      WB[q].at[0, pl.ds(0, CH)],
                                      ISEM[par]).wait()

        def issue_gather(q, par):
            pltpu.async_copy(h_hbm.at[DST[q]], RG[par], gsem)
            if gat:
                for hh in range(nh):
                    pltpu.async_copy(s1_hbms[hh].at[SRC[q]],
                                     S1G[hh][par].at[pl.ds(0, CH)], gsem)
                    pltpu.async_copy(s2_hbms[hh].at[DST[q]],
                                     S2G[hh][par].at[pl.ds(0, CH)], gsem)

        def wait_gather(q, par):
            pltpu.make_async_copy(h_hbm.at[DST[q]], RG[par], gsem).wait()
            if gat:
                for hh in range(nh):
                    pltpu.make_async_copy(s1_hbms[hh].at[SRC[q]],
                                          S1G[hh][par].at[pl.ds(0, CH)],
                                          gsem).wait()
                    pltpu.make_async_copy(s2_hbms[hh].at[DST[q]],
                                          S2G[hh][par].at[pl.ds(0, CH)],
                                          gsem).wait()

        def compute_scale(q, par):
            if gat:
                wb = WB[0]
                for v in range((CH + 15) // 16):
                    for hh in range(nh):
                        av = S1G[hh][par][pl.ds(v * 16, 16)]
                        bv = S2G[hh][par][pl.ds(v * 16, 16)]
                        cv = av + bv
                        ev = jnp.exp(-jnp.where(cv > 0, cv, _ALPHA * cv))
                        wb[hh, pl.ds(v * 16, 16)] = ev
            else:
                wb = WB[q]
            rg, rs = RG[par], RS[par]
            lane = lax.iota(jnp.int32, 16)

            def scale4(j4, _):
                for jj in range(4):
                    j = j4 * 4 + jj
                    ws = [wb[hh, pl.ds(j, 16)][0] for hh in range(nh)]
                    for hh in range(nh):
                        for f in range(Wh // 16):
                            c0 = hh * Wh + f * 16
                            rs[j, pl.ds(c0, 16)] = (
                                rg[j, pl.ds(c0, 16)] * ws[hh])
                    if gat:
                        ev = jnp.where(lane == 0, ws[0], 0.0)
                        if nh == 2:
                            ev = jnp.where(lane == 1, ws[1], ev)
                        rs[j, pl.ds(W, 16)] = ev
                return 0
            lax.fori_loop(0, CH // 4, scale4, 0)

        def issue_scatter(q, par):
            pltpu.async_copy(RS[par], acc.at[SRC[q]], SSEM[par], add=True)

        def wait_scatter(q, par):
            pltpu.make_async_copy(RS[par], acc.at[SRC[q]],
                                  SSEM[par]).wait()

        issue_idx(jnp.int32(0), 0, 0)
        issue_idx(jnp.int32(1), 1, 1)
        wait_idx(0, 0)
        issue_gather(0, 0)

        def block(k, _):
            for j in range(4):
                i = k * 4 + j
                q, par = j, j % 2

                @pl.when(i < NCHUNK)
                def _(i=i, q=q, par=par):
                    wait_gather(q, par)

                    @pl.when(i >= 2)
                    def _(i=i, q=q, par=par):
                        wait_scatter((q + 2) % 4, par)

                    @pl.when(i + 2 < NCHUNK)
                    def _(i=i, q=q, par=par):
                        issue_idx(i + 2, (q + 2) % 4, par)

                    compute_scale(q, par)
                    issue_scatter(q, par)

                    @pl.when(i + 1 < NCHUNK)
                    def _(i=i, q=q, par=par):
                        wait_idx((q + 1) % 4, (par + 1) % 2)
                        issue_gather((q + 1) % 4, (par + 1) % 2)
            return 0
        lax.fori_loop(0, NBLK, block, 0)

        wait_scatter((NCHUNK - 2) % 4, (NCHUNK - 2) % 2)
        wait_scatter((NCHUNK - 1) % 4, (NCHUNK - 1) % 2)

        plsc.subcore_barrier()

        for t in range(3):
            pltpu.sync_copy(acc.at[pl.ds(rbase + t * _ZR, _ZR)],
                            out_hbm.at[cid, pl.ds(rbase + t * _ZR, _ZR)])

        @pl.when(sid == _NS - 1)
        def _tail_copy():
            pltpu.sync_copy(acc.at[pl.ds(_NS * _RPT, 16)],
                            out_hbm.at[cid, pl.ds(_NS * _RPT, 16)])

    call = pl.kernel(
        body,
        out_type=jax.ShapeDtypeStruct((_NC, _N, WA), jnp.float32),
        mesh=mesh,
        scratch_types=scratch,
        compiler_params=pltpu.CompilerParams(use_tc_tiling_on_sc=False,
                                             needs_layout_passes=False),
    )
    if gat:
        return call(H, src, dst, *s1, *s2)
    return call(H, src, dst, wvals)


def _tc_snowball_mm(x, s_list, b_list, Wk, wout):
    nin = len(s_list)

    def body(*refs):
        x_ref = refs[0]
        s_refs = refs[1:1 + nin]
        b_refs = refs[1 + nin:1 + 2 * nin]
        w_ref = refs[1 + 2 * nin]
        o_ref = refs[-1]
        acc = jnp.dot(x_ref[...], w_ref[:_NFEAT],
                      preferred_element_type=jnp.float32)
        for j in range(nin):
            bj = jnp.tanh(s_refs[j][0] + s_refs[j][1] + b_refs[j][...])
            acc = acc + jnp.dot(
                bj, w_ref[_NFEAT + _NHID1 * j:_NFEAT + _NHID1 * (j + 1)],
                preferred_element_type=jnp.float32)
        o_ref[...] = acc

    in_specs = [pl.BlockSpec((_BLK, _NFEAT), lambda i: (i, 0))]
    for _ in range(nin):
        in_specs.append(pl.BlockSpec((2, _BLK, _NHID1), lambda i: (0, i, 0)))
    for _ in range(nin):
        in_specs.append(pl.BlockSpec((1, _NHID1), lambda i: (0, 0)))
    in_specs.append(pl.BlockSpec(Wk.shape, lambda i: (0, 0)))

    return pl.pallas_call(
        body,
        grid=(_GRID,),
        in_specs=in_specs,
        out_specs=pl.BlockSpec((_BLK, wout), lambda i: (i, 0)),
        out_shape=jax.ShapeDtypeStruct((_N, wout), jnp.float32),
    )(x, *s_list, *b_list, Wk)


def _tc_gat_proj(x, w_all, a_l, a_r):
    def body(x_ref, w_ref, al_ref, ar_ref, hp_ref, s1_ref, s2_ref):
        xb = x_ref[...]
        c1, c2 = [], []
        for p in range(4):
            hb = jnp.dot(xb, w_ref[:, 128 * p:128 * (p + 1)],
                         preferred_element_type=jnp.float32)
            hp_ref[p] = hb
            for hh in range(2):
                h = 2 * p + hh
                hv = hb[:, _NHID2 * hh:_NHID2 * (hh + 1)]
                c1.append(jnp.sum(hv * al_ref[h][None, :], axis=1))
                c2.append(jnp.sum(hv * ar_ref[h][None, :], axis=1))
        s1_ref[...] = jnp.stack(c1, axis=1)
        s2_ref[...] = jnp.stack(c2, axis=1)

    return pl.pallas_call(
        body,
        grid=(_GRID,),
        in_specs=[
            pl.BlockSpec((_BLK, _NFEAT), lambda i: (i, 0)),
            pl.BlockSpec((_NFEAT, _NHID2 * _NHEADS), lambda i: (0, 0)),
            pl.BlockSpec((_NHEADS, _NHID2), lambda i: (0, 0)),
            pl.BlockSpec((_NHEADS, _NHID2), lambda i: (0, 0)),
        ],
        out_specs=[
            pl.BlockSpec((4, _BLK, 128), lambda i: (0, i, 0)),
            pl.BlockSpec((_BLK, _NHEADS), lambda i: (i, 0)),
            pl.BlockSpec((_BLK, _NHEADS), lambda i: (i, 0)),
        ],
        out_shape=[
            jax.ShapeDtypeStruct((4, _N, 128), jnp.float32),
            jax.ShapeDtypeStruct((_N, _NHEADS), jnp.float32),
            jax.ShapeDtypeStruct((_N, _NHEADS), jnp.float32),
        ],
    )(x, w_all, a_l, a_r)


def _tc_gat_out(accs, oa_w, oa_l, oa_r):
    def body(*refs):
        a_refs = refs[:4]
        oaw_ref, oal_ref, oar_ref, g_ref, so_ref = refs[4:]
        g = jnp.zeros((_BLK, _NCLASS + 8), jnp.float32)
        for p in range(4):
            ab = a_refs[p]
            for hh in range(2):
                h = 2 * p + hh
                hp = (ab[0, :, _NHID2 * hh:_NHID2 * (hh + 1)]
                      + ab[1, :, _NHID2 * hh:_NHID2 * (hh + 1)])
                rs = ab[0, :, 128 + hh] + ab[1, :, 128 + hh]
                z0 = hp / (rs + 1e-15)[:, None]
                z0 = jnp.where(z0 > 0, z0, jnp.exp(z0) - 1.0)
                g = g + jnp.dot(
                    z0, oaw_ref[_NHID2 * h:_NHID2 * (h + 1)],
                    preferred_element_type=jnp.float32)
        g_ref[...] = g
        so_ref[...] = jnp.stack(
            [jnp.sum(g * oal_ref[...], axis=1),
             jnp.sum(g * oar_ref[...], axis=1)], axis=1)

    return pl.pallas_call(
        body,
        grid=(_GRID,),
        in_specs=[pl.BlockSpec((2, _BLK, 144), lambda i: (0, i, 0))] * 4
        + [
            pl.BlockSpec((_NHID2 * _NHEADS, _NCLASS + 8), lambda i: (0, 0)),
            pl.BlockSpec((1, _NCLASS + 8), lambda i: (0, 0)),
            pl.BlockSpec((1, _NCLASS + 8), lambda i: (0, 0)),
        ],
        out_specs=[
            pl.BlockSpec((_BLK, _NCLASS + 8), lambda i: (i, 0)),
            pl.BlockSpec((_BLK, 2), lambda i: (i, 0)),
        ],
        out_shape=[
            jax.ShapeDtypeStruct((_N, _NCLASS + 8), jnp.float32),
            jax.ShapeDtypeStruct((_N, 2), jnp.float32),
        ],
    )(*accs, oa_w, oa_l, oa_r)


def _tc_final(s4, out_b, go, idx_t, lab_f):
    NCP = _NCLASS + 8

    def body(s4_ref, ob_ref, go_ref, it_ref, lf_ref,
             y_ref, z_ref, l1_ref, l2_ref, cl_ref,
             ytr, ztr, ltr, clacc):
        i = pl.program_id(0)

        @pl.when(i == 0)
        def _init():
            ytr[...] = jnp.zeros_like(ytr)
            ztr[...] = jnp.zeros_like(ztr)
            ltr[...] = jnp.zeros_like(ltr)
            clacc[...] = jnp.zeros_like(clacc)

        col = lax.broadcasted_iota(jnp.int32, (_BLK, NCP), 1)
        mask = col < _NCLASS

        y_blk = s4_ref[0] + s4_ref[1] + ob_ref[...]
        rs = go_ref[0, :, NCP] + go_ref[1, :, NCP]
        zpre = (go_ref[0, :, :NCP] + go_ref[1, :, :NCP]) / (rs + 1e-15)[:, None]
        z_blk = jnp.where(zpre > 0, zpre, jnp.exp(zpre) - 1.0)
        y_ref[...] = y_blk
        z_ref[...] = z_blk

        ym = jnp.where(mask, y_blk, -1e30)
        my = jnp.max(ym, axis=1, keepdims=True)
        lsey = my + jnp.log(jnp.sum(jnp.exp(ym - my), axis=1, keepdims=True))
        logpy = ym - lsey
        zm = jnp.where(mask, z_blk, -1e30)
        mz = jnp.max(zm, axis=1, keepdims=True)
        pz = jnp.exp(zm - mz)
        pz = pz / jnp.sum(pz, axis=1, keepdims=True)
        clacc[...] += jnp.sum(jnp.where(mask, pz * (-logpy), 0.0)
                              ).reshape(1, 1)

        rows = i * _BLK + lax.broadcasted_iota(jnp.int32, (_NTRAIN, _BLK), 1)
        mf = jnp.where(it_ref[...] == rows, 1.0, 0.0)
        ytr[...] += jnp.dot(mf, y_blk, preferred_element_type=jnp.float32)
        ztr[...] += jnp.dot(mf, z_blk, preferred_element_type=jnp.float32)
        ltr[...] += jnp.dot(mf, lf_ref[...], preferred_element_type=jnp.float32)

        @pl.when(i == _GRID - 1)
        def _fin():
            tcol = lax.broadcasted_iota(jnp.int32, (_NTRAIN, NCP), 1)
            tmask = tcol < _NCLASS
            oh = jnp.where((tcol.astype(jnp.float32) == ltr[...]) & tmask,
                           1.0, 0.0)

            def ce(t_ref):
                tm = jnp.where(tmask, t_ref[...], -1e30)
                mt = jnp.max(tm, axis=1, keepdims=True)
                lse = mt + jnp.log(
                    jnp.sum(jnp.exp(tm - mt), axis=1, keepdims=True))
                lp = tm - lse
                return -jnp.sum(oh * lp) / _NTRAIN

            l1_ref[...] = ce(ytr).reshape(1, 1)
            l2_ref[...] = ce(ztr).reshape(1, 1)
            cl_ref[...] = clacc[...] / _N

    return pl.pallas_call(
        body,
        grid=(_GRID,),
        in_specs=[
            pl.BlockSpec((2, _BLK, NCP), lambda i: (0, i, 0)),
            pl.BlockSpec((1, NCP), lambda i: (0, 0)),
            pl.BlockSpec((2, _BLK, NCP + 16), lambda i: (0, i, 0)),
            pl.BlockSpec((_NTRAIN, 1), lambda i: (0, 0)),
            pl.BlockSpec((_BLK, 1), lambda i: (i, 0)),
        ],
        out_specs=[
            pl.BlockSpec((_BLK, NCP), lambda i: (i, 0)),
            pl.BlockSpec((_BLK, NCP), lambda i: (i, 0)),
            pl.BlockSpec((1, 1), lambda i: (0, 0)),
            pl.BlockSpec((1, 1), lambda i: (0, 0)),
            pl.BlockSpec((1, 1), lambda i: (0, 0)),
        ],
        out_shape=[
            jax.ShapeDtypeStruct((_N, NCP), jnp.float32),
            jax.ShapeDtypeStruct((_N, NCP), jnp.float32),
            jax.ShapeDtypeStruct((1, 1), jnp.float32),
            jax.ShapeDtypeStruct((1, 1), jnp.float32),
            jax.ShapeDtypeStruct((1, 1), jnp.float32),
        ],
        scratch_shapes=[
            pltpu.VMEM((_NTRAIN, NCP), jnp.float32),
            pltpu.VMEM((_NTRAIN, NCP), jnp.float32),
            pltpu.VMEM((_NTRAIN, 1), jnp.float32),
            pltpu.VMEM((1, 1), jnp.float32),
        ],
    )(s4, out_b, go, idx_t, lab_f)


def kernel(x, params, edge_weight, edge_index, idx_train, labels):
    src = edge_index[0]
    dst = edge_index[1]

    s_parts = []
    b_list = []
    for k in range(_NLAYERS):
        p_k = _tc_snowball_mm(x, s_parts, b_list,
                              params['snow_W_%d' % k], _NHID1)
        s_parts.append(_sc_spmm(p_k, src, dst, wvals=edge_weight))
        b_list.append(params['snow_b_%d' % k].reshape(1, _NHID1))

    w_out = jnp.pad(params['out_W'], ((0, 0), (0, 8)))
    p_out = _tc_snowball_mm(x, s_parts, b_list, w_out, _NCLASS + 8)
    s4 = _sc_spmm(p_out, src, dst, wvals=edge_weight)

    w_all = jnp.concatenate(
        [params['att_W_%d' % h] for h in range(_NHEADS)], axis=1)
    a_l = jnp.stack([params['att_a_%d' % h][0, :_NHID2]
                     for h in range(_NHEADS)])
    a_r = jnp.stack([params['att_a_%d' % h][0, _NHID2:]
                     for h in range(_NHEADS)])
    hp, s1, s2 = _tc_gat_proj(x, w_all, a_l, a_r)
    s1t = s1.T
    s2t = s2.T
    accs = [
        _sc_spmm(hp[p], src, dst,
                 s1=[s1t[2 * p], s1t[2 * p + 1]],
                 s2=[s2t[2 * p], s2t[2 * p + 1]], nh=2)
        for p in range(4)
    ]

    oa_w = jnp.pad(params['outatt_W'], ((0, 0), (0, 8)))
    oa_l = jnp.pad(params['outatt_a'][0, :_NCLASS], (0, 8))[None]
    oa_r = jnp.pad(params['outatt_a'][0, _NCLASS:], (0, 8))[None]
    g, so = _tc_gat_out(accs, oa_w, oa_l, oa_r)
    sot = so.T
    go = _sc_spmm(g, src, dst, s1=[sot[0]], s2=[sot[1]], nh=1)

    out_b = jnp.pad(params['out_b'], (0, 8))[None]
    y48, z48, l1, l2, cl = _tc_final(
        s4, out_b, go,
        idx_train.reshape(_NTRAIN, 1),
        labels.astype(jnp.float32).reshape(_N, 1))

    y = y48[:, :_NCLASS]
    z = z48[:, :_NCLASS]
    l1s = l1[0, 0]
    l2s = l2[0, 0]
    cls = cl[0, 0]
    return (y, z, l1s, l2s,
            l1s + jnp.float32(0.05) * cls, l2s + jnp.float32(0.05) * cls)
